# baseline (speedup 1.0000x reference)
import functools

import jax
import jax.numpy as jnp
from jax import lax
from jax.experimental import pallas as pl
from jax.experimental.pallas import tpu as pltpu
from jax.experimental.pallas import tpu_sc as plsc

N = 10000
E = 320000
D = 128
R = 4

NPAD = 10240
NPASS = 4
RANGE = NPAD // NPASS
SLOTS = R * RANGE
TRASH = SLOTS
ACC_ROWS = 10496
NC = 2
NS = 16
NT = NC * NS
EPT = E // NT
NV = EPT // 16
C = 128
NBUF = 2
SEL_ROWS = 80
MASK14 = (1 << 14) - 1


def _scan_body(epk, sel_out, nch_out, e_all, sel, nch_buf):
    ci = lax.axis_index("c")
    si = lax.axis_index("s")
    tid = ci * NS + si
    pltpu.sync_copy(epk.at[pl.ds(tid * EPT, EPT)], e_all)

    it16 = lax.iota(jnp.int32, 16)
    nch_v = jnp.zeros((16,), jnp.int32)
    for rho in range(NPASS):
        lo = rho * RANGE

        def scan_step(i, cnt):
            p = e_all[pl.ds(i * 16, 16)]
            src_v = jnp.bitwise_and(p, MASK14)
            dst_v = jnp.bitwise_and(jnp.right_shift(p, 14), MASK14)
            et_v = jnp.right_shift(p, 28)
            m = (dst_v >= lo) & (dst_v < lo + RANGE)
            slot_v = et_v * RANGE + (dst_v - lo)
            entry = jnp.bitwise_or(src_v, jnp.left_shift(slot_v, 14))
            mi = m.astype(jnp.int32)
            pf = plsc.cumsum(mi)
            q = cnt + pf - 1
            ri = jnp.right_shift(q, 7)
            co = jnp.bitwise_and(q, 127)
            plsc.store_scatter(sel, [ri, co], entry, mask=m)
            return cnt + jnp.sum(mi)

        cnt = lax.fori_loop(0, NV, scan_step, jnp.int32(0))

        trash_v = jnp.full((16,), TRASH << 14, jnp.int32)
        for k in range(C // 16):
            q = cnt + k * 16 + it16
            ri = jnp.right_shift(q, 7)
            co = jnp.bitwise_and(q, 127)
            plsc.store_scatter(sel, [ri, co], trash_v)

        nch = (cnt + (C - 1)) // C
        nch_v = jnp.where(it16 == rho, nch, nch_v)
        pltpu.sync_copy(sel, sel_out.at[tid, rho])

    nch_buf[pl.ds(0, 16)] = nch_v
    pltpu.sync_copy(nch_buf, nch_out.at[tid])


@functools.lru_cache(maxsize=None)
def _make_scan():
    return pl.kernel(
        _scan_body,
        out_type=(
            jax.ShapeDtypeStruct((NT, NPASS, SEL_ROWS, 128), jnp.int32),
            jax.ShapeDtypeStruct((NT, 16), jnp.int32),
        ),
        mesh=_mesh(),
        scratch_types=[
            pltpu.VMEM((EPT,), jnp.int32),
            pltpu.VMEM((SEL_ROWS, 128), jnp.int32),
            pltpu.VMEM((16,), jnp.int32),
        ],
        compiler_params=pltpu.CompilerParams(needs_layout_passes=False),
    )


def _proc_body(with_cnt, h, sel_in, nch_in, z640, o128, *rest):
    if with_cnt:
        a_out, cnt_out = rest[0], rest[1]
        scr = rest[2:]
    else:
        a_out = rest[0]
        scr = rest[1:]
    sel = scr[0]
    src_c = scr[1:1 + NBUF]
    slot_c = scr[1 + NBUF:1 + 2 * NBUF]
    rows = scr[1 + 2 * NBUF:1 + 3 * NBUF]
    nch_buf = scr[1 + 3 * NBUF]
    cline = scr[2 + 3 * NBUF]
    acc = scr[3 + 3 * NBUF]
    sem = scr[4 + 3 * NBUF:4 + 4 * NBUF]
    sem_s = scr[4 + 4 * NBUF:4 + 5 * NBUF]

    ci = lax.axis_index("c")
    si = lax.axis_index("s")
    tid = ci * NS + si

    pltpu.sync_copy(nch_in.at[tid], nch_buf)
    it16 = lax.iota(jnp.int32, 16)
    nch_v = nch_buf[pl.ds(0, 16)]

    def unpack(j, b, slot_only=False):
        for k in range(C // 16):
            pp = sel[j, pl.ds(k * 16, 16)]
            if not slot_only:
                src_c[b][pl.ds(k * 16, 16)] = jnp.bitwise_and(pp, MASK14)
            slot_c[b][pl.ds(k * 16, 16)] = jnp.right_shift(pp, 14)

    def run_chunks(nch):
        for p in range(NBUF - 1):
            @pl.when(p < nch)
            def _(p=p):
                unpack(p, p)
                pltpu.async_copy(h.at[src_c[p]], rows[p], sem[p])

        def outer(t, carry):
            for b in range(NBUF):
                j = NBUF * t + b
                bn = (b + NBUF - 1) % NBUF

                @pl.when(j < nch)
                def _(j=j, b=b, bn=bn):
                    @pl.when(j + NBUF - 1 < nch)
                    def _():
                        @pl.when(j >= 1)
                        def _():
                            pltpu.make_async_copy(rows[bn],
                                                  acc.at[slot_c[bn]],
                                                  sem_s[bn]).wait()
                        unpack(j + NBUF - 1, bn)
                        pltpu.async_copy(h.at[src_c[bn]], rows[bn], sem[bn])
                    pltpu.make_async_copy(h.at[src_c[b]], rows[b],
                                          sem[b]).wait()
                    pltpu.async_copy(rows[b], acc.at[slot_c[b]], sem_s[b],
                                     add=True)
            return carry

        lax.fori_loop(0, (nch + NBUF - 1) // NBUF, outer, jnp.int32(0))

        for b in range(NBUF):
            @pl.when(nch > b)
            def _(b=b):
                pltpu.make_async_copy(rows[b], acc.at[slot_c[b]],
                                      sem_s[b]).wait()

    def run_cnt_chunks(nch):
        def outer(t, carry):
            for b in range(NBUF):
                j = NBUF * t + b

                @pl.when(j < nch)
                def _(j=j, b=b):
                    @pl.when(j >= NBUF)
                    def _():
                        pltpu.make_async_copy(rows[0], acc.at[slot_c[b]],
                                              sem[b]).wait()
                    unpack(j, b, slot_only=True)
                    pltpu.async_copy(rows[0], acc.at[slot_c[b]], sem[b],
                                     add=True)
            return carry

        lax.fori_loop(0, (nch + NBUF - 1) // NBUF, outer, jnp.int32(0))

        for b in range(NBUF):
            @pl.when(nch > b)
            def _(b=b):
                pltpu.make_async_copy(rows[0], acc.at[slot_c[b]],
                                      sem[b]).wait()

    def writeout(dst_ref, lo):
        r_idx = si // 4
        noff = (si % 4) * 640
        pltpu.sync_copy(acc.at[pl.ds(si * 640, 640)],
                        dst_ref.at[ci, r_idx, pl.ds(lo + noff, 640)])

    for rho in range(NPASS):
        nch = jnp.sum(jnp.where(it16 == rho, nch_v, 0))
        pltpu.sync_copy(z640, acc.at[pl.ds(si * 640, 640)])
        plsc.subcore_barrier()
        pltpu.sync_copy(sel_in.at[tid, rho], sel)
        run_chunks(nch)
        plsc.subcore_barrier()
        writeout(a_out, rho * RANGE)

    if with_cnt:
        pltpu.sync_copy(o128, rows[0])
        for rho in range(NPASS):
            nch = jnp.sum(jnp.where(it16 == rho, nch_v, 0))
            pltpu.sync_copy(z640, acc.at[pl.ds(si * 640, 640)])
            plsc.subcore_barrier()
            pltpu.sync_copy(sel_in.at[tid, rho], sel)
            run_cnt_chunks(nch)
            plsc.subcore_barrier()
            base = si * 640
            for seg in range(640 // C):
                pltpu.sync_copy(acc.at[pl.ds(base + seg * C, C)], rows[1])
                for k in range(C // 16):
                    vals = plsc.load_gather(
                        rows[1], [k * 16 + it16, jnp.zeros((16,), jnp.int32)])
                    cline[pl.ds(seg * C + k * 16, 16)] = vals
            pltpu.sync_copy(cline, cnt_out.at[ci, rho, pl.ds(base, 640)])


def _mesh():
    return plsc.VectorSubcoreMesh(core_axis_name="c", subcore_axis_name="s",
                                  num_cores=NC, num_subcores=NS)


@functools.lru_cache(maxsize=None)
def _make_proc(with_cnt):
    out_type = [jax.ShapeDtypeStruct((NC, R, NPAD, D), jnp.float32)]
    if with_cnt:
        out_type.append(jax.ShapeDtypeStruct((NC, NPASS, SLOTS), jnp.float32))
    return pl.kernel(
        functools.partial(_proc_body, with_cnt),
        out_type=tuple(out_type),
        mesh=_mesh(),
        scratch_types=(
            [pltpu.VMEM((SEL_ROWS, 128), jnp.int32)]
            + [pltpu.VMEM((C,), jnp.int32) for _ in range(NBUF)]
            + [pltpu.VMEM((C,), jnp.int32) for _ in range(NBUF)]
            + [pltpu.VMEM((C, D), jnp.float32) for _ in range(NBUF)]
            + [pltpu.VMEM((16,), jnp.int32)]
            + [pltpu.VMEM((640,), jnp.float32)]
            + [pltpu.VMEM_SHARED((ACC_ROWS, D), jnp.float32)]
            + [pltpu.SemaphoreType.DMA for _ in range(2 * NBUF)]
        ),
        compiler_params=pltpu.CompilerParams(needs_layout_passes=False),
    )


TB = 512


def _tc_body(h_ref, a_ref, cnt_ref, wrel_ref, wroot_ref, bias_ref, g_ref,
             be_ref, o_ref):
    nsub = TB // 128
    bb = pl.program_id(0) % (RANGE // TB)
    h = h_ref[...]
    out = jnp.dot(h, wroot_ref[...], preferred_element_type=jnp.float32)
    out = out + bias_ref[...]
    c4 = (cnt_ref[0, 0, :, pl.ds(bb * nsub, nsub), :]
          + cnt_ref[1, 0, :, pl.ds(bb * nsub, nsub), :])
    invT = jnp.transpose(1.0 / jnp.maximum(c4.reshape(R * nsub, 128), 1.0))
    for r in range(R):
        a = a_ref[0, r] + a_ref[1, r]
        inv = jnp.concatenate(
            [invT[:, r * nsub + s:r * nsub + s + 1] for s in range(nsub)],
            axis=0)
        out = out + jnp.dot(a * inv, wrel_ref[r],
                            preferred_element_type=jnp.float32)
    mu = jnp.mean(out, axis=1, keepdims=True)
    xc = out - mu
    var = jnp.mean(xc * xc, axis=1, keepdims=True)
    y = xc * lax.rsqrt(var + 1e-5)
    y = y * g_ref[...] + be_ref[...]
    o_ref[...] = jnp.maximum(y, 0.0)


def _tc_layer(h, a_p, cnt_p, wrel, wroot, bias, g, be):
    nb = NPAD // TB
    return pl.pallas_call(
        _tc_body,
        grid=(nb,),
        in_specs=[
            pl.BlockSpec((TB, D), lambda b: (b, 0)),
            pl.BlockSpec((NC, R, TB, D), lambda b: (0, 0, b, 0)),
            pl.BlockSpec((NC, 1, R, RANGE // 128, 128),
                         lambda b: (0, b // (RANGE // TB), 0, 0, 0)),
            pl.BlockSpec((R, D, D), lambda b: (0, 0, 0)),
            pl.BlockSpec((D, D), lambda b: (0, 0)),
            pl.BlockSpec((1, D), lambda b: (0, 0)),
            pl.BlockSpec((1, D), lambda b: (0, 0)),
            pl.BlockSpec((1, D), lambda b: (0, 0)),
        ],
        out_specs=pl.BlockSpec((TB, D), lambda b: (b, 0)),
        out_shape=jax.ShapeDtypeStruct((NPAD, D), jnp.float32),
    )(h, a_p, cnt_p, wrel, wroot, bias, g, be)


def kernel(x, edge_index, edge_attr, Wrel0, Wroot0, b0, g0, be0,
           Wrel1, Wroot1, b1, g1, be1):
    src = edge_index[0]
    dst = edge_index[1]
    et = edge_attr[:, 0]
    epk = src | (dst << 14) | (et << 28)

    h0 = jnp.pad(x, ((0, NPAD - N), (0, 0)))
    z640 = jnp.zeros((640, D), jnp.float32)
    o128 = jnp.ones((C, D), jnp.float32)

    sel_s, nch_s = _make_scan()(epk)
    a0, cnt_c = _make_proc(True)(h0, sel_s, nch_s, z640, o128)
    cnt_p = cnt_c.reshape(NC, NPASS, R, RANGE // 128, 128)

    h1 = _tc_layer(h0, a0, cnt_p, Wrel0, Wroot0, b0.reshape(1, D),
                   g0.reshape(1, D), be0.reshape(1, D))
    (a1,) = _make_proc(False)(h1, sel_s, nch_s, z640, o128)
    h2 = _tc_layer(h1, a1, cnt_p, Wrel1, Wroot1, b1.reshape(1, D),
                   g1.reshape(1, D), be1.reshape(1, D))
    return jnp.stack([h1[:N], h2[:N]])


# speedup vs baseline: 1.4381x; 1.4381x over previous
import functools

import jax
import jax.numpy as jnp
from jax import lax
from jax.experimental import pallas as pl
from jax.experimental.pallas import tpu as pltpu
from jax.experimental.pallas import tpu_sc as plsc

N = 10000
E = 320000
D = 128
R = 4

NPAD = 10240
NPASS = 4
RANGE = NPAD // NPASS
SLOTS = R * RANGE
TRASH = SLOTS
ACC_ROWS = 10496
NC = 2
NS = 16
NT = NC * NS
EPT = E // NT
NV = EPT // 16
C = 64
NBUF = 4
SEL_ROWS = 80
MASK14 = (1 << 14) - 1


def _scan_body(epk, sel_out, nch_out, e_all, sel, nch_buf):
    ci = lax.axis_index("c")
    si = lax.axis_index("s")
    tid = ci * NS + si
    pltpu.sync_copy(epk.at[pl.ds(tid * EPT, EPT)], e_all)

    it16 = lax.iota(jnp.int32, 16)
    nch_v = jnp.zeros((16,), jnp.int32)
    for rho in range(NPASS):
        lo = rho * RANGE

        def scan_step(i, cnt):
            p = e_all[pl.ds(i * 16, 16)]
            src_v = jnp.bitwise_and(p, MASK14)
            dst_v = jnp.bitwise_and(jnp.right_shift(p, 14), MASK14)
            et_v = jnp.right_shift(p, 28)
            m = (dst_v >= lo) & (dst_v < lo + RANGE)
            slot_v = et_v * RANGE + (dst_v - lo)
            entry = jnp.bitwise_or(src_v, jnp.left_shift(slot_v, 14))
            mi = m.astype(jnp.int32)
            pf = plsc.cumsum(mi)
            q = cnt + pf - 1
            ri = jnp.right_shift(q, 7)
            co = jnp.bitwise_and(q, 127)
            plsc.store_scatter(sel, [ri, co], entry, mask=m)
            return cnt + jnp.sum(mi)

        cnt = lax.fori_loop(0, NV, scan_step, jnp.int32(0))

        trash_v = jnp.full((16,), TRASH << 14, jnp.int32)
        for k in range(C // 16):
            q = cnt + k * 16 + it16
            ri = jnp.right_shift(q, 7)
            co = jnp.bitwise_and(q, 127)
            plsc.store_scatter(sel, [ri, co], trash_v)

        nch = (cnt + (C - 1)) // C
        nch_v = jnp.where(it16 == rho, nch, nch_v)
        pltpu.sync_copy(sel, sel_out.at[tid, rho])

    nch_buf[pl.ds(0, 16)] = nch_v
    pltpu.sync_copy(nch_buf, nch_out.at[tid])


@functools.lru_cache(maxsize=None)
def _make_scan():
    return pl.kernel(
        _scan_body,
        out_type=(
            jax.ShapeDtypeStruct((NT, NPASS, SEL_ROWS, 128), jnp.int32),
            jax.ShapeDtypeStruct((NT, 16), jnp.int32),
        ),
        mesh=_mesh(),
        scratch_types=[
            pltpu.VMEM((EPT,), jnp.int32),
            pltpu.VMEM((SEL_ROWS, 128), jnp.int32),
            pltpu.VMEM((16,), jnp.int32),
        ],
        compiler_params=pltpu.CompilerParams(needs_layout_passes=False),
    )


def _proc_body(with_cnt, h, sel_in, nch_in, z640, o128, *rest):
    if with_cnt:
        a_out, cnt_out = rest[0], rest[1]
        scr = rest[2:]
    else:
        a_out = rest[0]
        scr = rest[1:]
    sel = scr[0]
    src_c = scr[1:1 + NBUF]
    slot_c = scr[1 + NBUF:1 + 2 * NBUF]
    rows = scr[1 + 2 * NBUF:1 + 3 * NBUF]
    nch_buf = scr[1 + 3 * NBUF]
    cline = scr[2 + 3 * NBUF]
    acc = scr[3 + 3 * NBUF]
    sem = scr[4 + 3 * NBUF:4 + 4 * NBUF]
    sem_s = scr[4 + 4 * NBUF:4 + 5 * NBUF]

    ci = lax.axis_index("c")
    si = lax.axis_index("s")
    tid = ci * NS + si

    pltpu.sync_copy(nch_in.at[tid], nch_buf)
    it16 = lax.iota(jnp.int32, 16)
    nch_v = nch_buf[pl.ds(0, 16)]

    def unpack(j, b, slot_only=False):
        row = jnp.right_shift(j, 1)
        cb = jnp.bitwise_and(j, 1) * C
        for k in range(C // 16):
            pp = sel[row, pl.ds(cb + k * 16, 16)]
            if not slot_only:
                src_c[b][pl.ds(k * 16, 16)] = jnp.bitwise_and(pp, MASK14)
            slot_c[b][pl.ds(k * 16, 16)] = jnp.right_shift(pp, 14)

    def run_chunks(nch):
        for p in range(NBUF - 1):
            @pl.when(p < nch)
            def _(p=p):
                unpack(p, p)
                pltpu.async_copy(h.at[src_c[p]], rows[p], sem[p])

        def outer(t, carry):
            for b in range(NBUF):
                j = NBUF * t + b
                bn = (b + NBUF - 1) % NBUF

                @pl.when(j < nch)
                def _(j=j, b=b, bn=bn):
                    @pl.when(j + NBUF - 1 < nch)
                    def _():
                        @pl.when(j >= 1)
                        def _():
                            pltpu.make_async_copy(rows[bn],
                                                  acc.at[slot_c[bn]],
                                                  sem_s[bn]).wait()
                        unpack(j + NBUF - 1, bn)
                        pltpu.async_copy(h.at[src_c[bn]], rows[bn], sem[bn])
                    pltpu.make_async_copy(h.at[src_c[b]], rows[b],
                                          sem[b]).wait()
                    pltpu.async_copy(rows[b], acc.at[slot_c[b]], sem_s[b],
                                     add=True)
            return carry

        lax.fori_loop(0, (nch + NBUF - 1) // NBUF, outer, jnp.int32(0))

        for b in range(NBUF):
            @pl.when(nch > b)
            def _(b=b):
                pltpu.make_async_copy(rows[b], acc.at[slot_c[b]],
                                      sem_s[b]).wait()

    def run_cnt_chunks(nch):
        def outer(t, carry):
            for b in range(NBUF):
                j = NBUF * t + b

                @pl.when(j < nch)
                def _(j=j, b=b):
                    @pl.when(j >= NBUF)
                    def _():
                        pltpu.make_async_copy(rows[0], acc.at[slot_c[b]],
                                              sem[b]).wait()
                    unpack(j, b, slot_only=True)
                    pltpu.async_copy(rows[0], acc.at[slot_c[b]], sem[b],
                                     add=True)
            return carry

        lax.fori_loop(0, (nch + NBUF - 1) // NBUF, outer, jnp.int32(0))

        for b in range(NBUF):
            @pl.when(nch > b)
            def _(b=b):
                pltpu.make_async_copy(rows[0], acc.at[slot_c[b]],
                                      sem[b]).wait()

    def writeout(dst_ref, lo):
        r_idx = si // 4
        noff = (si % 4) * 640
        pltpu.sync_copy(acc.at[pl.ds(si * 640, 640)],
                        dst_ref.at[ci, r_idx, pl.ds(lo + noff, 640)])

    for rho in range(NPASS):
        nch = jnp.sum(jnp.where(it16 == rho, nch_v, 0))
        pltpu.sync_copy(z640, acc.at[pl.ds(si * 640, 640)])
        plsc.subcore_barrier()
        pltpu.sync_copy(sel_in.at[tid, rho], sel)
        run_chunks(nch)
        plsc.subcore_barrier()
        writeout(a_out, rho * RANGE)

    if with_cnt:
        pltpu.sync_copy(o128, rows[0])
        for rho in range(NPASS):
            nch = jnp.sum(jnp.where(it16 == rho, nch_v, 0))
            pltpu.sync_copy(z640, acc.at[pl.ds(si * 640, 640)])
            plsc.subcore_barrier()
            pltpu.sync_copy(sel_in.at[tid, rho], sel)
            run_cnt_chunks(nch)
            plsc.subcore_barrier()
            base = si * 640
            for seg in range(640 // C):
                pltpu.sync_copy(acc.at[pl.ds(base + seg * C, C)], rows[1])
                for k in range(C // 16):
                    vals = plsc.load_gather(
                        rows[1], [k * 16 + it16, jnp.zeros((16,), jnp.int32)])
                    cline[pl.ds(seg * C + k * 16, 16)] = vals
            pltpu.sync_copy(cline, cnt_out.at[ci, rho, pl.ds(base, 640)])


def _mesh():
    return plsc.VectorSubcoreMesh(core_axis_name="c", subcore_axis_name="s",
                                  num_cores=NC, num_subcores=NS)


@functools.lru_cache(maxsize=None)
def _make_proc(with_cnt):
    out_type = [jax.ShapeDtypeStruct((NC, R, NPAD, D), jnp.float32)]
    if with_cnt:
        out_type.append(jax.ShapeDtypeStruct((NC, NPASS, SLOTS), jnp.float32))
    return pl.kernel(
        functools.partial(_proc_body, with_cnt),
        out_type=tuple(out_type),
        mesh=_mesh(),
        scratch_types=(
            [pltpu.VMEM((SEL_ROWS, 128), jnp.int32)]
            + [pltpu.VMEM((C,), jnp.int32) for _ in range(NBUF)]
            + [pltpu.VMEM((C,), jnp.int32) for _ in range(NBUF)]
            + [pltpu.VMEM((C, D), jnp.float32) for _ in range(NBUF)]
            + [pltpu.VMEM((16,), jnp.int32)]
            + [pltpu.VMEM((640,), jnp.float32)]
            + [pltpu.VMEM_SHARED((ACC_ROWS, D), jnp.float32)]
            + [pltpu.SemaphoreType.DMA for _ in range(2 * NBUF)]
        ),
        compiler_params=pltpu.CompilerParams(needs_layout_passes=False),
    )


TB = 512


def _tc_body(h_ref, a_ref, cnt_ref, wrel_ref, wroot_ref, bias_ref, g_ref,
             be_ref, o_ref):
    nsub = TB // 128
    bb = pl.program_id(0) % (RANGE // TB)
    h = h_ref[...]
    out = jnp.dot(h, wroot_ref[...], preferred_element_type=jnp.float32)
    out = out + bias_ref[...]
    c4 = (cnt_ref[0, 0, :, pl.ds(bb * nsub, nsub), :]
          + cnt_ref[1, 0, :, pl.ds(bb * nsub, nsub), :])
    invT = jnp.transpose(1.0 / jnp.maximum(c4.reshape(R * nsub, 128), 1.0))
    for r in range(R):
        a = a_ref[0, r] + a_ref[1, r]
        inv = jnp.concatenate(
            [invT[:, r * nsub + s:r * nsub + s + 1] for s in range(nsub)],
            axis=0)
        out = out + jnp.dot(a * inv, wrel_ref[r],
                            preferred_element_type=jnp.float32)
    mu = jnp.mean(out, axis=1, keepdims=True)
    xc = out - mu
    var = jnp.mean(xc * xc, axis=1, keepdims=True)
    y = xc * lax.rsqrt(var + 1e-5)
    y = y * g_ref[...] + be_ref[...]
    o_ref[...] = jnp.maximum(y, 0.0)


def _tc_layer(h, a_p, cnt_p, wrel, wroot, bias, g, be):
    nb = NPAD // TB
    return pl.pallas_call(
        _tc_body,
        grid=(nb,),
        in_specs=[
            pl.BlockSpec((TB, D), lambda b: (b, 0)),
            pl.BlockSpec((NC, R, TB, D), lambda b: (0, 0, b, 0)),
            pl.BlockSpec((NC, 1, R, RANGE // 128, 128),
                         lambda b: (0, b // (RANGE // TB), 0, 0, 0)),
            pl.BlockSpec((R, D, D), lambda b: (0, 0, 0)),
            pl.BlockSpec((D, D), lambda b: (0, 0)),
            pl.BlockSpec((1, D), lambda b: (0, 0)),
            pl.BlockSpec((1, D), lambda b: (0, 0)),
            pl.BlockSpec((1, D), lambda b: (0, 0)),
        ],
        out_specs=pl.BlockSpec((TB, D), lambda b: (b, 0)),
        out_shape=jax.ShapeDtypeStruct((NPAD, D), jnp.float32),
    )(h, a_p, cnt_p, wrel, wroot, bias, g, be)


def kernel(x, edge_index, edge_attr, Wrel0, Wroot0, b0, g0, be0,
           Wrel1, Wroot1, b1, g1, be1):
    src = edge_index[0]
    dst = edge_index[1]
    et = edge_attr[:, 0]
    epk = src | (dst << 14) | (et << 28)

    h0 = jnp.pad(x, ((0, NPAD - N), (0, 0)))
    z640 = jnp.zeros((640, D), jnp.float32)
    o128 = jnp.ones((C, D), jnp.float32)

    sel_s, nch_s = _make_scan()(epk)
    a0, cnt_c = _make_proc(True)(h0, sel_s, nch_s, z640, o128)
    cnt_p = cnt_c.reshape(NC, NPASS, R, RANGE // 128, 128)

    h1 = _tc_layer(h0, a0, cnt_p, Wrel0, Wroot0, b0.reshape(1, D),
                   g0.reshape(1, D), be0.reshape(1, D))
    (a1,) = _make_proc(False)(h1, sel_s, nch_s, z640, o128)
    h2 = _tc_layer(h1, a1, cnt_p, Wrel1, Wroot1, b1.reshape(1, D),
                   g1.reshape(1, D), be1.reshape(1, D))
    return jnp.stack([h1[:N], h2[:N]])
